# baseline (speedup 1.0000x reference)
import functools

import jax
import jax.numpy as jnp
from jax import lax
from jax.experimental import pallas as pl
from jax.experimental.pallas import tpu as pltpu
from jax.experimental.pallas import tpu_sc as plsc

NC, NS, L = 2, 16, 16
NT = NC * NS
P_PAD = 1_048_576
ROWS = P_PAD // 128
NB = 40_000
NB_PAD = 40_064
NF = 16_896
C = 128
CQ = 32
NQ = 4

A_WIN_ROWS = 8
A_WINS = 32
B_CHUNK_ROWS = 4
B_CHUNKS = 128
B_SUB = 128
B_TROWS = B_CHUNKS * B_CHUNK_ROWS


def _mesh():
    return plsc.VectorSubcoreMesh(
        core_axis_name="c", subcore_axis_name="s",
        num_cores=NC, num_subcores=NS)


@functools.partial(
    pl.kernel,
    out_type=jax.ShapeDtypeStruct((ROWS, 128), jnp.float32),
    mesh=_mesh(),
    compiler_params=pltpu.CompilerParams(needs_layout_passes=False, use_tc_tiling_on_sc=False),
    scratch_types=[
        pltpu.VMEM((A_WIN_ROWS, 128), jnp.int32),
        pltpu.VMEM((A_WIN_ROWS, 128), jnp.float32),
        pltpu.SemaphoreType.DMA,
    ],
)
def _gather_depth(depth_hbm, rd_hbm, gd_hbm, idx_v, val_v, sem):
    wid = lax.axis_index("s") * NC + lax.axis_index("c")
    row0 = wid * (A_WINS * A_WIN_ROWS)

    def win(w, _):
        r0 = row0 + w * A_WIN_ROWS
        pltpu.sync_copy(rd_hbm.at[pl.ds(r0, A_WIN_ROWS)], idx_v)
        cps = [pltpu.async_copy(depth_hbm.at[idx_v.at[j]], val_v.at[j], sem)
               for j in range(A_WIN_ROWS)]
        for cp in cps:
            cp.wait()
        pltpu.sync_copy(val_v, gd_hbm.at[pl.ds(r0, A_WIN_ROWS)])
        return 0

    lax.fori_loop(0, A_WINS, win, 0)


@functools.partial(
    pl.kernel,
    out_type=jax.ShapeDtypeStruct((NQ, NB_PAD, CQ), jnp.float32),
    mesh=_mesh(),
    compiler_params=pltpu.CompilerParams(needs_layout_passes=False, use_tc_tiling_on_sc=False),
    scratch_types=[
        pltpu.VMEM_SHARED((NB_PAD, CQ), jnp.float32),
        pltpu.VMEM_SHARED((NF, CQ), jnp.float32),
        pltpu.VMEM((B_CHUNK_ROWS, 128), jnp.int32),
        pltpu.VMEM((B_CHUNK_ROWS, 128), jnp.int32),
        pltpu.VMEM((B_CHUNK_ROWS, 128), jnp.int32),
        pltpu.VMEM((B_CHUNK_ROWS, 128), jnp.int32),
        pltpu.VMEM((B_CHUNK_ROWS * 128,), jnp.float32),
        pltpu.VMEM((B_CHUNK_ROWS * 128,), jnp.float32),
        pltpu.VMEM((B_SUB, CQ), jnp.float32),
        pltpu.VMEM((B_SUB, CQ), jnp.float32),
        pltpu.VMEM((B_SUB, CQ), jnp.float32),
        pltpu.SemaphoreType.DMA,
        pltpu.SemaphoreType.DMA,
        pltpu.SemaphoreType.DMA,
        pltpu.SemaphoreType.DMA,
        pltpu.SemaphoreType.DMA,
        pltpu.SemaphoreType.DMA,
        pltpu.SemaphoreType.DMA,
        pltpu.SemaphoreType.DMA,
    ],
)
def _pool(ftab_hbm, rf_hbm, rb_hbm, gd_hbm, zeros_hbm, out_hbm,
          acc, ftab_s, rfA, rfB, rbA, rbB, gdA, gdB,
          fr0, fr1, fr2, gsem0, gsem1, gsem2, ssem0, ssem1, ssem2,
          isemA, isemB):
    c = lax.axis_index("c")
    s = lax.axis_index("s")
    zrows = NB_PAD // NS
    frows = NF // NS
    base = s * B_TROWS
    frb = (fr0, fr1, fr2)
    gsems = (gsem0, gsem1, gsem2)
    ssems = (ssem0, ssem1, ssem2)
    sets = ((rfA, rbA, gdA, isemA), (rfB, rbB, gdB, isemB))

    def fire_idx(r0, st):
        rf_v, rb_v, gd_v, isem = st
        return [
            pltpu.async_copy(rf_hbm.at[pl.ds(r0, B_CHUNK_ROWS)], rf_v, isem),
            pltpu.async_copy(rb_hbm.at[pl.ds(r0, B_CHUNK_ROWS)], rb_v, isem),
            pltpu.async_copy(gd_hbm.at[pl.ds(r0 * 128, B_CHUNK_ROWS * 128)],
                             gd_v, isem),
        ]

    def mul_sub(fr_v, gd_v, jj):
        def mul16(bb, _):
            p0 = bb * L
            d16 = gd_v[pl.ds(jj * B_SUB + p0, L)]
            for l in range(L):
                dl = lax.broadcast_in_dim(d16[l], (L,), ())
                i = p0 + l
                fr_v[i, pl.ds(0, L)] = fr_v[i, pl.ds(0, L)] * dl
                fr_v[i, pl.ds(L, L)] = fr_v[i, pl.ds(L, L)] * dl
            return 0
        lax.fori_loop(0, B_SUB // L, mul16, 0)

    NSUB = 2 * B_CHUNK_ROWS

    for rnd in range(NQ // NC):
        q = c * (NQ // NC) + rnd
        pltpu.sync_copy(zeros_hbm, acc.at[pl.ds(s * zrows, zrows)])
        pltpu.sync_copy(ftab_hbm.at[q].at[pl.ds(s * frows, frows)],
                        ftab_s.at[pl.ds(s * frows, frows)])
        plsc.subcore_barrier()

        for h in fire_idx(base, sets[0]):
            h.wait()

        def pair(g, _):
            r2g = base + 2 * g * B_CHUNK_ROWS
            hiB = fire_idx(r2g + B_CHUNK_ROWS, sets[1])
            hiA = None
            hg = {}
            hs = {}
            hg[0] = pltpu.async_copy(ftab_s.at[rfA.at[0]], frb[0], gsems[0])
            for j in range(NSUB):
                if j >= 2:
                    hs[j - 2].wait()
                if j == B_CHUNK_ROWS - 1:
                    for h in hiB:
                        h.wait()
                if j == NSUB - 2:
                    hiA = fire_idx(
                        jnp.minimum(r2g + 2 * B_CHUNK_ROWS,
                                    base + B_TROWS - B_CHUNK_ROWS), sets[0])
                if j + 1 < NSUB:
                    nrf = sets[(j + 1) // B_CHUNK_ROWS][0]
                    nb = (j + 1) % 3
                    hg[j + 1] = pltpu.async_copy(
                        ftab_s.at[nrf.at[(j + 1) % B_CHUNK_ROWS]],
                        frb[nb], gsems[nb])
                hg[j].wait()
                st = sets[j // B_CHUNK_ROWS]
                jj = j % B_CHUNK_ROWS
                mul_sub(frb[j % 3], st[2], jj)
                hs[j] = pltpu.async_copy(frb[j % 3], acc.at[st[1].at[jj]],
                                         ssems[j % 3], add=True)
            for j in range(NSUB - 2, NSUB):
                hs[j].wait()
            for h in hiA:
                h.wait()
            return 0

        lax.fori_loop(0, B_CHUNKS // 2, pair, 0)
        plsc.subcore_barrier()
        pltpu.sync_copy(acc.at[pl.ds(s * zrows, zrows)],
                        out_hbm.at[q].at[pl.ds(s * zrows, zrows)])
        plsc.subcore_barrier()


def kernel(ranks_depth, ranks_feat, ranks_bev, n_points, depth, feat):
    P = ranks_depth.shape[0]
    depth_flat = depth.reshape(-1)
    dsz = depth_flat.shape[0]
    ftab = feat.reshape(NF, NQ, CQ).transpose(1, 0, 2)

    pad = P_PAD - P
    ar = jnp.arange(pad, dtype=jnp.int32)
    rd = jnp.concatenate([ranks_depth, ar % dsz]).reshape(ROWS, 128)
    rf = jnp.concatenate([ranks_feat, ar % NF]).reshape(ROWS, 128)
    rb = jnp.concatenate([ranks_bev, ar % NB]).reshape(ROWS, 128)

    gd = _gather_depth(depth_flat, rd).reshape(-1)
    valid = jnp.arange(P_PAD, dtype=jnp.int32) < n_points
    gd = jnp.where(valid, gd, 0.0)

    zeros = jnp.zeros((NB_PAD // NS, CQ), jnp.float32)
    out4 = _pool(ftab, rf, rb, gd, zeros)
    out = out4[:, :NB, :].transpose(1, 0, 2).reshape(NB, C)
    return out.reshape(1, 1, 200, 200, C)


# speedup vs baseline: 1.0546x; 1.0546x over previous
import functools

import jax
import jax.numpy as jnp
from jax import lax
from jax.experimental import pallas as pl
from jax.experimental.pallas import tpu as pltpu
from jax.experimental.pallas import tpu_sc as plsc

NC, NS, L = 2, 16, 16
NT = NC * NS
P_PAD = 1_048_576
ROWS = P_PAD // 128
NB = 40_000
NB_PAD = 40_064
NF = 16_896
C = 128
CQ = 32
NQ = 4

A_WIN_ROWS = 8
A_WINS = 32
B_CHUNK_ROWS = 4
B_CHUNKS = 128
B_SUB = 128
B_TROWS = B_CHUNKS * B_CHUNK_ROWS


def _mesh():
    return plsc.VectorSubcoreMesh(
        core_axis_name="c", subcore_axis_name="s",
        num_cores=NC, num_subcores=NS)


DSZ = 1_993_728


@functools.partial(
    pl.kernel,
    out_type=jax.ShapeDtypeStruct((P_PAD,), jnp.float32),
    mesh=_mesh(),
    compiler_params=pltpu.CompilerParams(needs_layout_passes=False, use_tc_tiling_on_sc=False),
    scratch_types=[
        pltpu.VMEM_SHARED((DSZ,), jnp.float32),
        pltpu.VMEM((A_WIN_ROWS, 128), jnp.int32),
        pltpu.VMEM((A_WIN_ROWS * 128,), jnp.float32),
        pltpu.SemaphoreType.DMA,
    ],
)
def _gather_depth(depth_hbm, rd_hbm, gd_hbm, depth_s, idx_v, val_v, sem):
    c = lax.axis_index("c")
    s = lax.axis_index("s")
    wid = s * NC + c
    drows = DSZ // NS
    pltpu.sync_copy(depth_hbm.at[pl.ds(s * drows, drows)],
                    depth_s.at[pl.ds(s * drows, drows)])
    plsc.subcore_barrier()
    row0 = wid * (A_WINS * A_WIN_ROWS)

    def win(w, _):
        r0 = row0 + w * A_WIN_ROWS
        pltpu.sync_copy(rd_hbm.at[pl.ds(r0, A_WIN_ROWS)], idx_v)
        cps = [pltpu.async_copy(depth_s.at[idx_v.at[j]],
                                val_v.at[pl.ds(j * 128, 128)], sem)
               for j in range(A_WIN_ROWS)]
        for cp in cps:
            cp.wait()
        pltpu.sync_copy(val_v, gd_hbm.at[pl.ds(r0 * 128, A_WIN_ROWS * 128)])
        return 0

    lax.fori_loop(0, A_WINS, win, 0)


@functools.partial(
    pl.kernel,
    out_type=jax.ShapeDtypeStruct((NQ, NB_PAD, CQ), jnp.float32),
    mesh=_mesh(),
    compiler_params=pltpu.CompilerParams(needs_layout_passes=False, use_tc_tiling_on_sc=False),
    scratch_types=[
        pltpu.VMEM_SHARED((NB_PAD, CQ), jnp.float32),
        pltpu.VMEM_SHARED((NF, CQ), jnp.float32),
        pltpu.VMEM((B_CHUNK_ROWS, 128), jnp.int32),
        pltpu.VMEM((B_CHUNK_ROWS, 128), jnp.int32),
        pltpu.VMEM((B_CHUNK_ROWS, 128), jnp.int32),
        pltpu.VMEM((B_CHUNK_ROWS, 128), jnp.int32),
        pltpu.VMEM((B_CHUNK_ROWS * 128,), jnp.float32),
        pltpu.VMEM((B_CHUNK_ROWS * 128,), jnp.float32),
        pltpu.VMEM((B_SUB, CQ), jnp.float32),
        pltpu.VMEM((B_SUB, CQ), jnp.float32),
        pltpu.VMEM((B_SUB, CQ), jnp.float32),
        pltpu.SemaphoreType.DMA,
        pltpu.SemaphoreType.DMA,
        pltpu.SemaphoreType.DMA,
        pltpu.SemaphoreType.DMA,
        pltpu.SemaphoreType.DMA,
        pltpu.SemaphoreType.DMA,
        pltpu.SemaphoreType.DMA,
        pltpu.SemaphoreType.DMA,
    ],
)
def _pool(ftab_hbm, rf_hbm, rb_hbm, gd_hbm, zeros_hbm, out_hbm,
          acc, ftab_s, rfA, rfB, rbA, rbB, gdA, gdB,
          fr0, fr1, fr2, gsem0, gsem1, gsem2, ssem0, ssem1, ssem2,
          isemA, isemB):
    c = lax.axis_index("c")
    s = lax.axis_index("s")
    zrows = NB_PAD // NS
    frows = NF // NS
    base = s * B_TROWS
    frb = (fr0, fr1, fr2)
    gsems = (gsem0, gsem1, gsem2)
    ssems = (ssem0, ssem1, ssem2)
    sets = ((rfA, rbA, gdA, isemA), (rfB, rbB, gdB, isemB))

    def fire_idx(r0, st):
        rf_v, rb_v, gd_v, isem = st
        return [
            pltpu.async_copy(rf_hbm.at[pl.ds(r0, B_CHUNK_ROWS)], rf_v, isem),
            pltpu.async_copy(rb_hbm.at[pl.ds(r0, B_CHUNK_ROWS)], rb_v, isem),
            pltpu.async_copy(gd_hbm.at[pl.ds(r0 * 128, B_CHUNK_ROWS * 128)],
                             gd_v, isem),
        ]

    def mul_sub(fr_v, gd_v, jj):
        def mul16(bb, _):
            p0 = bb * L
            d16 = gd_v[pl.ds(jj * B_SUB + p0, L)]
            for l in range(L):
                dl = lax.broadcast_in_dim(d16[l], (L,), ())
                i = p0 + l
                fr_v[i, pl.ds(0, L)] = fr_v[i, pl.ds(0, L)] * dl
                fr_v[i, pl.ds(L, L)] = fr_v[i, pl.ds(L, L)] * dl
            return 0
        lax.fori_loop(0, B_SUB // L, mul16, 0)

    NSUB = 2 * B_CHUNK_ROWS

    for rnd in range(NQ // NC):
        q = c * (NQ // NC) + rnd
        pltpu.sync_copy(zeros_hbm, acc.at[pl.ds(s * zrows, zrows)])
        pltpu.sync_copy(ftab_hbm.at[q].at[pl.ds(s * frows, frows)],
                        ftab_s.at[pl.ds(s * frows, frows)])
        plsc.subcore_barrier()

        for h in fire_idx(base, sets[0]):
            h.wait()

        def pair(g, _):
            r2g = base + 2 * g * B_CHUNK_ROWS
            hiB = fire_idx(r2g + B_CHUNK_ROWS, sets[1])
            hiA = None
            hg = {}
            hs = {}
            hg[0] = pltpu.async_copy(ftab_s.at[rfA.at[0]], frb[0], gsems[0])
            for j in range(NSUB):
                if j >= 2:
                    hs[j - 2].wait()
                if j == B_CHUNK_ROWS - 1:
                    for h in hiB:
                        h.wait()
                if j == NSUB - 2:
                    hiA = fire_idx(
                        jnp.minimum(r2g + 2 * B_CHUNK_ROWS,
                                    base + B_TROWS - B_CHUNK_ROWS), sets[0])
                if j + 1 < NSUB:
                    nrf = sets[(j + 1) // B_CHUNK_ROWS][0]
                    nb = (j + 1) % 3
                    hg[j + 1] = pltpu.async_copy(
                        ftab_s.at[nrf.at[(j + 1) % B_CHUNK_ROWS]],
                        frb[nb], gsems[nb])
                hg[j].wait()
                st = sets[j // B_CHUNK_ROWS]
                jj = j % B_CHUNK_ROWS
                mul_sub(frb[j % 3], st[2], jj)
                hs[j] = pltpu.async_copy(frb[j % 3], acc.at[st[1].at[jj]],
                                         ssems[j % 3], add=True)
            for j in range(NSUB - 2, NSUB):
                hs[j].wait()
            for h in hiA:
                h.wait()
            return 0

        lax.fori_loop(0, B_CHUNKS // 2, pair, 0)
        plsc.subcore_barrier()
        pltpu.sync_copy(acc.at[pl.ds(s * zrows, zrows)],
                        out_hbm.at[q].at[pl.ds(s * zrows, zrows)])
        plsc.subcore_barrier()


def kernel(ranks_depth, ranks_feat, ranks_bev, n_points, depth, feat):
    P = ranks_depth.shape[0]
    depth_flat = depth.reshape(-1)
    ftab = feat.reshape(NF, NQ, CQ).transpose(1, 0, 2)

    pad = P_PAD - P
    ar = jnp.arange(pad, dtype=jnp.int32)
    arp = jnp.arange(P, dtype=jnp.int32)
    rb_real = jnp.where(arp < n_points, ranks_bev, NB + (arp % (NB_PAD - NB)))
    rd = jnp.concatenate([ranks_depth, ar % DSZ]).reshape(ROWS, 128)
    rf = jnp.concatenate([ranks_feat, ar % NF]).reshape(ROWS, 128)
    rb = jnp.concatenate([rb_real, NB + (ar % (NB_PAD - NB))]).reshape(ROWS, 128)

    gd = _gather_depth(depth_flat, rd)

    zeros = jnp.zeros((NB_PAD // NS, CQ), jnp.float32)
    out4 = _pool(ftab, rf, rb, gd, zeros)
    out = out4[:, :NB, :].transpose(1, 0, 2).reshape(NB, C)
    return out.reshape(1, 1, 200, 200, C)


# speedup vs baseline: 1.0864x; 1.0302x over previous
import functools

import jax
import jax.numpy as jnp
from jax import lax
from jax.experimental import pallas as pl
from jax.experimental.pallas import tpu as pltpu
from jax.experimental.pallas import tpu_sc as plsc

NC, NS, L = 2, 16, 16
NT = NC * NS
P_PAD = 1_048_576
ROWS = P_PAD // 128
NB = 40_000
NB_PAD = 40_064
NF = 16_896
C = 128
CQ = 32
NQ = 4

A_WIN_ROWS = 8
A_WINS = 32
B_CHUNK_ROWS = 4
B_CHUNKS = 128
B_SUB = 128
B_TROWS = B_CHUNKS * B_CHUNK_ROWS


def _mesh():
    return plsc.VectorSubcoreMesh(
        core_axis_name="c", subcore_axis_name="s",
        num_cores=NC, num_subcores=NS)


DSZ = 1_993_728


@functools.partial(
    pl.kernel,
    out_type=jax.ShapeDtypeStruct((P_PAD,), jnp.float32),
    mesh=_mesh(),
    compiler_params=pltpu.CompilerParams(needs_layout_passes=False, use_tc_tiling_on_sc=False),
    scratch_types=[
        pltpu.VMEM_SHARED((DSZ,), jnp.float32),
        pltpu.VMEM((A_WIN_ROWS, 128), jnp.int32),
        pltpu.VMEM((A_WIN_ROWS * 128,), jnp.float32),
        pltpu.SemaphoreType.DMA,
    ],
)
def _gather_depth(depth_hbm, rd_hbm, gd_hbm, depth_s, idx_v, val_v, sem):
    c = lax.axis_index("c")
    s = lax.axis_index("s")
    wid = s * NC + c
    drows = DSZ // NS
    pltpu.sync_copy(depth_hbm.at[pl.ds(s * drows, drows)],
                    depth_s.at[pl.ds(s * drows, drows)])
    plsc.subcore_barrier()
    row0 = wid * (A_WINS * A_WIN_ROWS)

    def win(w, _):
        r0 = row0 + w * A_WIN_ROWS
        pltpu.sync_copy(rd_hbm.at[pl.ds(r0, A_WIN_ROWS)], idx_v)
        cps = [pltpu.async_copy(depth_s.at[idx_v.at[j]],
                                val_v.at[pl.ds(j * 128, 128)], sem)
               for j in range(A_WIN_ROWS)]
        for cp in cps:
            cp.wait()
        pltpu.sync_copy(val_v, gd_hbm.at[pl.ds(r0 * 128, A_WIN_ROWS * 128)])
        return 0

    lax.fori_loop(0, A_WINS, win, 0)


@functools.partial(
    pl.kernel,
    out_type=jax.ShapeDtypeStruct((NQ, NB_PAD, CQ), jnp.float32),
    mesh=_mesh(),
    compiler_params=pltpu.CompilerParams(needs_layout_passes=False, use_tc_tiling_on_sc=False),
    scratch_types=[
        pltpu.VMEM_SHARED((NB_PAD, CQ), jnp.float32),
        pltpu.VMEM_SHARED((NF, CQ), jnp.float32),
        pltpu.VMEM((B_CHUNK_ROWS, 128), jnp.int32),
        pltpu.VMEM((B_CHUNK_ROWS, 128), jnp.int32),
        pltpu.VMEM((B_CHUNK_ROWS, 128), jnp.int32),
        pltpu.VMEM((B_CHUNK_ROWS, 128), jnp.int32),
        pltpu.VMEM((B_CHUNK_ROWS * 128,), jnp.float32),
        pltpu.VMEM((B_CHUNK_ROWS * 128,), jnp.float32),
        pltpu.VMEM((B_SUB, CQ), jnp.float32),
        pltpu.VMEM((B_SUB, CQ), jnp.float32),
        pltpu.VMEM((B_SUB, CQ), jnp.float32),
        pltpu.SemaphoreType.DMA,
        pltpu.SemaphoreType.DMA,
        pltpu.SemaphoreType.DMA,
        pltpu.SemaphoreType.DMA,
        pltpu.SemaphoreType.DMA,
        pltpu.SemaphoreType.DMA,
        pltpu.SemaphoreType.DMA,
        pltpu.SemaphoreType.DMA,
    ],
)
def _pool(ftab_hbm, rf_hbm, rb_hbm, gd_hbm, zeros_hbm, out_hbm,
          acc, ftab_s, rfA, rfB, rbA, rbB, gdA, gdB,
          fr0, fr1, fr2, gsem0, gsem1, gsem2, ssem0, ssem1, ssem2,
          isemA, isemB):
    c = lax.axis_index("c")
    s = lax.axis_index("s")
    zrows = NB_PAD // NS
    frows = NF // NS
    base = s * B_TROWS
    frb = (fr0, fr1, fr2)
    gsems = (gsem0, gsem1, gsem2)
    ssems = (ssem0, ssem1, ssem2)
    sets = ((rfA, rbA, gdA, isemA), (rfB, rbB, gdB, isemB))

    def fire_idx(r0, st):
        rf_v, rb_v, gd_v, isem = st
        return [
            pltpu.async_copy(rf_hbm.at[pl.ds(r0, B_CHUNK_ROWS)], rf_v, isem),
            pltpu.async_copy(rb_hbm.at[pl.ds(r0, B_CHUNK_ROWS)], rb_v, isem),
            pltpu.async_copy(gd_hbm.at[pl.ds(r0 * 128, B_CHUNK_ROWS * 128)],
                             gd_v, isem),
        ]

    def mul_sub(fr_v, gd_v, jj):
        def mul16(bb, _):
            p0 = bb * L
            d16 = gd_v[pl.ds(jj * B_SUB + p0, L)]
            for l in range(L):
                dl = lax.broadcast_in_dim(d16[l], (L,), ())
                i = p0 + l
                fr_v[i, pl.ds(0, L)] = fr_v[i, pl.ds(0, L)] * dl
                fr_v[i, pl.ds(L, L)] = fr_v[i, pl.ds(L, L)] * dl
            return 0
        lax.fori_loop(0, B_SUB // L, mul16, 0)

    NSUB = 2 * B_CHUNK_ROWS

    for rnd in range(NQ // NC):
        q = c * (NQ // NC) + rnd
        pltpu.sync_copy(zeros_hbm, acc.at[pl.ds(s * zrows, zrows)])
        pltpu.sync_copy(ftab_hbm.at[pl.ds(s * frows, frows), q],
                        ftab_s.at[pl.ds(s * frows, frows)])
        plsc.subcore_barrier()

        for h in fire_idx(base, sets[0]):
            h.wait()

        def pair(g, _):
            r2g = base + 2 * g * B_CHUNK_ROWS
            hiB = fire_idx(r2g + B_CHUNK_ROWS, sets[1])
            hiA = None
            hg = {}
            hs = {}
            hg[0] = pltpu.async_copy(ftab_s.at[rfA.at[0]], frb[0], gsems[0])
            for j in range(NSUB):
                if j >= 2:
                    hs[j - 2].wait()
                if j == B_CHUNK_ROWS - 1:
                    for h in hiB:
                        h.wait()
                if j == NSUB - 2:
                    hiA = fire_idx(
                        jnp.minimum(r2g + 2 * B_CHUNK_ROWS,
                                    base + B_TROWS - B_CHUNK_ROWS), sets[0])
                if j + 1 < NSUB:
                    nrf = sets[(j + 1) // B_CHUNK_ROWS][0]
                    nb = (j + 1) % 3
                    hg[j + 1] = pltpu.async_copy(
                        ftab_s.at[nrf.at[(j + 1) % B_CHUNK_ROWS]],
                        frb[nb], gsems[nb])
                hg[j].wait()
                st = sets[j // B_CHUNK_ROWS]
                jj = j % B_CHUNK_ROWS
                mul_sub(frb[j % 3], st[2], jj)
                hs[j] = pltpu.async_copy(frb[j % 3], acc.at[st[1].at[jj]],
                                         ssems[j % 3], add=True)
            for j in range(NSUB - 2, NSUB):
                hs[j].wait()
            for h in hiA:
                h.wait()
            return 0

        lax.fori_loop(0, B_CHUNKS // 2, pair, 0)
        plsc.subcore_barrier()
        pltpu.sync_copy(acc.at[pl.ds(s * zrows, zrows)],
                        out_hbm.at[q].at[pl.ds(s * zrows, zrows)])
        plsc.subcore_barrier()


def kernel(ranks_depth, ranks_feat, ranks_bev, n_points, depth, feat):
    P = ranks_depth.shape[0]
    depth_flat = depth.reshape(-1)
    ftab = feat.reshape(NF, NQ, CQ)

    pad = P_PAD - P
    ar = jnp.arange(pad, dtype=jnp.int32)
    arp = jnp.arange(P, dtype=jnp.int32)
    rb_real = jnp.where(arp < n_points, ranks_bev, NB + (arp % (NB_PAD - NB)))
    rd = jnp.concatenate([ranks_depth, ar % DSZ]).reshape(ROWS, 128)
    rf = jnp.concatenate([ranks_feat, ar % NF]).reshape(ROWS, 128)
    rb = jnp.concatenate([rb_real, NB + (ar % (NB_PAD - NB))]).reshape(ROWS, 128)

    gd = _gather_depth(depth_flat, rd)

    zeros = jnp.zeros((NB_PAD // NS, CQ), jnp.float32)
    out4 = _pool(ftab, rf, rb, gd, zeros)
    out = out4[:, :NB, :].transpose(1, 0, 2).reshape(NB, C)
    return out.reshape(1, 1, 200, 200, C)


# speedup vs baseline: 1.0948x; 1.0078x over previous
import functools

import jax
import jax.numpy as jnp
from jax import lax
from jax.experimental import pallas as pl
from jax.experimental.pallas import tpu as pltpu
from jax.experimental.pallas import tpu_sc as plsc

NC, NS, L = 2, 16, 16
NT = NC * NS
P_PAD = 1_048_576
ROWS = P_PAD // 128
NB = 40_000
NB_PAD = 40_064
NF = 16_896
C = 128
CQ = 32
NQ = 4

A_WIN_ROWS = 8
A_WINS = 32
B_CHUNK_ROWS = 4
B_CHUNKS = 128
B_SUB = 128
B_TROWS = B_CHUNKS * B_CHUNK_ROWS


def _mesh():
    return plsc.VectorSubcoreMesh(
        core_axis_name="c", subcore_axis_name="s",
        num_cores=NC, num_subcores=NS)


DSZ = 1_993_728


@functools.partial(
    pl.kernel,
    out_type=jax.ShapeDtypeStruct((P_PAD,), jnp.float32),
    mesh=_mesh(),
    compiler_params=pltpu.CompilerParams(needs_layout_passes=False, use_tc_tiling_on_sc=False),
    scratch_types=[
        pltpu.VMEM_SHARED((DSZ,), jnp.float32),
        pltpu.VMEM((A_WIN_ROWS, 128), jnp.int32),
        pltpu.VMEM((A_WIN_ROWS * 128,), jnp.float32),
        pltpu.SemaphoreType.DMA,
    ],
)
def _gather_depth(depth_hbm, rd_hbm, gd_hbm, depth_s, idx_v, val_v, sem):
    c = lax.axis_index("c")
    s = lax.axis_index("s")
    wid = s * NC + c
    drows = DSZ // NS
    pltpu.sync_copy(depth_hbm.at[pl.ds(s * drows, drows)],
                    depth_s.at[pl.ds(s * drows, drows)])
    plsc.subcore_barrier()
    row0 = wid * (A_WINS * A_WIN_ROWS)

    def win(w, _):
        r0 = row0 + w * A_WIN_ROWS
        pltpu.sync_copy(rd_hbm.at[pl.ds(r0, A_WIN_ROWS)], idx_v)
        cps = [pltpu.async_copy(depth_s.at[idx_v.at[j]],
                                val_v.at[pl.ds(j * 128, 128)], sem)
               for j in range(A_WIN_ROWS)]
        for cp in cps:
            cp.wait()
        pltpu.sync_copy(val_v, gd_hbm.at[pl.ds(r0 * 128, A_WIN_ROWS * 128)])
        return 0

    lax.fori_loop(0, A_WINS, win, 0)


@functools.partial(
    pl.kernel,
    out_type=jax.ShapeDtypeStruct((NQ, NB_PAD, CQ), jnp.float32),
    mesh=_mesh(),
    compiler_params=pltpu.CompilerParams(needs_layout_passes=False, use_tc_tiling_on_sc=False),
    scratch_types=[
        pltpu.VMEM_SHARED((NB_PAD, CQ), jnp.float32),
        pltpu.VMEM_SHARED((NF, CQ), jnp.float32),
        pltpu.VMEM((B_CHUNK_ROWS, 128), jnp.int32),
        pltpu.VMEM((B_CHUNK_ROWS, 128), jnp.int32),
        pltpu.VMEM((B_CHUNK_ROWS, 128), jnp.int32),
        pltpu.VMEM((B_CHUNK_ROWS, 128), jnp.int32),
        pltpu.VMEM((B_CHUNK_ROWS * 128,), jnp.float32),
        pltpu.VMEM((B_CHUNK_ROWS * 128,), jnp.float32),
        pltpu.VMEM((B_SUB, CQ), jnp.float32),
        pltpu.VMEM((B_SUB, CQ), jnp.float32),
        pltpu.VMEM((B_SUB, CQ), jnp.float32),
        pltpu.SemaphoreType.DMA,
        pltpu.SemaphoreType.DMA,
        pltpu.SemaphoreType.DMA,
        pltpu.SemaphoreType.DMA,
        pltpu.SemaphoreType.DMA,
        pltpu.SemaphoreType.DMA,
        pltpu.SemaphoreType.DMA,
        pltpu.SemaphoreType.DMA,
    ],
)
def _pool(ftab_hbm, rf_hbm, rb_hbm, gd_hbm, zeros_hbm, out_hbm,
          acc, ftab_s, rfA, rfB, rbA, rbB, gdA, gdB,
          fr0, fr1, fr2, gsem0, gsem1, gsem2, ssem0, ssem1, ssem2,
          isemA, isemB):
    c = lax.axis_index("c")
    s = lax.axis_index("s")
    zrows = NB_PAD // NS
    frows = NF // NS
    base = s * B_TROWS
    frb = (fr0, fr1, fr2)
    gsems = (gsem0, gsem1, gsem2)
    ssems = (ssem0, ssem1, ssem2)
    sets = ((rfA, rbA, gdA, isemA), (rfB, rbB, gdB, isemB))

    def fire_idx(r0, st):
        rf_v, rb_v, gd_v, isem = st
        return [
            pltpu.async_copy(rf_hbm.at[pl.ds(r0, B_CHUNK_ROWS)], rf_v, isem),
            pltpu.async_copy(rb_hbm.at[pl.ds(r0, B_CHUNK_ROWS)], rb_v, isem),
            pltpu.async_copy(gd_hbm.at[pl.ds(r0 * 128, B_CHUNK_ROWS * 128)],
                             gd_v, isem),
        ]

    def mul_sub(fr_v, gd_v, jj):
        def mul32(bb, _):
            for h in range(2):
                p0 = bb * 2 * L + h * L
                d16 = gd_v[pl.ds(jj * B_SUB + p0, L)]
                for l in range(L):
                    dl = lax.broadcast_in_dim(d16[l], (L,), ())
                    i = p0 + l
                    fr_v[i, pl.ds(0, L)] = fr_v[i, pl.ds(0, L)] * dl
                    fr_v[i, pl.ds(L, L)] = fr_v[i, pl.ds(L, L)] * dl
            return 0
        lax.fori_loop(0, B_SUB // (2 * L), mul32, 0)

    NSUB = 2 * B_CHUNK_ROWS

    for rnd in range(NQ // NC):
        q = c * (NQ // NC) + rnd
        pltpu.sync_copy(zeros_hbm, acc.at[pl.ds(s * zrows, zrows)])
        pltpu.sync_copy(ftab_hbm.at[pl.ds(s * frows, frows), q],
                        ftab_s.at[pl.ds(s * frows, frows)])
        plsc.subcore_barrier()

        for h in fire_idx(base, sets[0]):
            h.wait()

        def pair(g, _):
            r2g = base + 2 * g * B_CHUNK_ROWS
            hiB = fire_idx(r2g + B_CHUNK_ROWS, sets[1])
            hiA = None
            hg = {}
            hs = {}
            hg[0] = pltpu.async_copy(ftab_s.at[rfA.at[0]], frb[0], gsems[0])
            for j in range(NSUB):
                if j >= 2:
                    hs[j - 2].wait()
                if j == B_CHUNK_ROWS - 1:
                    for h in hiB:
                        h.wait()
                if j == NSUB - 2:
                    hiA = fire_idx(
                        jnp.minimum(r2g + 2 * B_CHUNK_ROWS,
                                    base + B_TROWS - B_CHUNK_ROWS), sets[0])
                if j + 1 < NSUB:
                    nrf = sets[(j + 1) // B_CHUNK_ROWS][0]
                    nb = (j + 1) % 3
                    hg[j + 1] = pltpu.async_copy(
                        ftab_s.at[nrf.at[(j + 1) % B_CHUNK_ROWS]],
                        frb[nb], gsems[nb])
                hg[j].wait()
                st = sets[j // B_CHUNK_ROWS]
                jj = j % B_CHUNK_ROWS
                mul_sub(frb[j % 3], st[2], jj)
                hs[j] = pltpu.async_copy(frb[j % 3], acc.at[st[1].at[jj]],
                                         ssems[j % 3], add=True)
            for j in range(NSUB - 2, NSUB):
                hs[j].wait()
            for h in hiA:
                h.wait()
            return 0

        lax.fori_loop(0, B_CHUNKS // 2, pair, 0)
        plsc.subcore_barrier()
        pltpu.sync_copy(acc.at[pl.ds(s * zrows, zrows)],
                        out_hbm.at[q].at[pl.ds(s * zrows, zrows)])
        plsc.subcore_barrier()


def kernel(ranks_depth, ranks_feat, ranks_bev, n_points, depth, feat):
    P = ranks_depth.shape[0]
    depth_flat = depth.reshape(-1)
    ftab = feat.reshape(NF, NQ, CQ)

    pad = P_PAD - P
    ar = jnp.arange(pad, dtype=jnp.int32)
    arp = jnp.arange(P, dtype=jnp.int32)
    rb_real = jnp.where(arp < n_points, ranks_bev, NB + (arp % (NB_PAD - NB)))
    rd = jnp.concatenate([ranks_depth, ar % DSZ]).reshape(ROWS, 128)
    rf = jnp.concatenate([ranks_feat, ar % NF]).reshape(ROWS, 128)
    rb = jnp.concatenate([rb_real, NB + (ar % (NB_PAD - NB))]).reshape(ROWS, 128)

    gd = _gather_depth(depth_flat, rd)

    zeros = jnp.zeros((NB_PAD // NS, CQ), jnp.float32)
    out4 = _pool(ftab, rf, rb, gd, zeros)
    out = out4[:, :NB, :].transpose(1, 0, 2).reshape(NB, C)
    return out.reshape(1, 1, 200, 200, C)
